# baseline (speedup 1.0000x reference)
import functools

import jax
import jax.numpy as jnp
from jax import lax
from jax.experimental import pallas as pl
from jax.experimental.pallas import tpu as pltpu
from jax.experimental.pallas import tpu_sc as plsc


_CHUNK = 128


def _make_gather(B, DU, DM, NC, NS):
    NW = NC * NS
    b_per_w = B // NW
    n_chunks = b_per_w // _CHUNK
    mesh = plsc.VectorSubcoreMesh(core_axis_name="c", subcore_axis_name="s")

    @functools.partial(
        pl.kernel,
        mesh=mesh,
        compiler_params=pltpu.CompilerParams(use_tc_tiling_on_sc=False),
        out_type=[
            jax.ShapeDtypeStruct((B, DU), jnp.float32),
            jax.ShapeDtypeStruct((B, DM), jnp.float32),
        ],
        scratch_types=[
            pltpu.VMEM((n_chunks, _CHUNK), jnp.int32),
            pltpu.VMEM((n_chunks, _CHUNK), jnp.int32),
            pltpu.VMEM((b_per_w, DU), jnp.float32),
            pltpu.VMEM((b_per_w, DM), jnp.float32),
            pltpu.SemaphoreType.DMA,
        ],
    )
    def gather_kernel(uidx_hbm, midx_hbm, uemb_hbm, memb_hbm,
                      outu_hbm, outm_hbm,
                      uidx_v, midx_v, urows_v, mrows_v, sem):
        wid = lax.axis_index("s") * NC + lax.axis_index("c")
        base = wid * b_per_w
        row0 = wid * n_chunks
        pltpu.sync_copy(uidx_hbm.at[pl.ds(row0, n_chunks)], uidx_v)
        pltpu.sync_copy(midx_hbm.at[pl.ds(row0, n_chunks)], midx_v)
        copies = []
        for c in range(n_chunks):
            copies.append(pltpu.async_copy(
                uemb_hbm.at[uidx_v.at[c]],
                urows_v.at[pl.ds(c * _CHUNK, _CHUNK)], sem))
            copies.append(pltpu.async_copy(
                memb_hbm.at[midx_v.at[c]],
                mrows_v.at[pl.ds(c * _CHUNK, _CHUNK)], sem))
        for cp in copies:
            cp.wait()
        pltpu.sync_copy(urows_v, outu_hbm.at[pl.ds(base, b_per_w)])
        pltpu.sync_copy(mrows_v, outm_hbm.at[pl.ds(base, b_per_w)])

    return gather_kernel


def _mlp_body(xu_ref, xm_ref, W1_ref, b1_ref, W2_ref, b2_ref,
              Wout_ref, bout_ref, out_ref):
    xu = xu_ref[...]
    xm = xm_ref[...]
    W1 = W1_ref[...]
    DU = xu.shape[1]
    h = (jnp.dot(xu, W1[:DU], preferred_element_type=jnp.float32,
                 precision=lax.Precision.HIGHEST)
         + jnp.dot(xm, W1[DU:], preferred_element_type=jnp.float32,
                   precision=lax.Precision.HIGHEST)
         + b1_ref[...])
    h = jnp.maximum(h, 0.0)
    h = jnp.dot(h, W2_ref[...], preferred_element_type=jnp.float32,
                precision=lax.Precision.HIGHEST) + b2_ref[...]
    h = jnp.maximum(h, 0.0)
    out_ref[...] = (jnp.dot(h, Wout_ref[...],
                            preferred_element_type=jnp.float32,
                            precision=lax.Precision.HIGHEST)
                    + bout_ref[...])


def _run_mlp(xu, xm, W1, b1, W2, b2, Wout, bout):
    B, DU = xu.shape
    DM = xm.shape[1]
    H1 = W1.shape[1]
    H2 = W2.shape[1]
    BM = 2048
    grid = (B // BM,)
    const = lambda shape: pl.BlockSpec(shape, lambda i: (0,) * len(shape))
    return pl.pallas_call(
        _mlp_body,
        grid=grid,
        in_specs=[
            pl.BlockSpec((BM, DU), lambda i: (i, 0)),
            pl.BlockSpec((BM, DM), lambda i: (i, 0)),
            const((DU + DM, H1)),
            const((1, H1)),
            const((H1, H2)),
            const((1, H2)),
            const((H2, 1)),
            const((1, 1)),
        ],
        out_specs=pl.BlockSpec((BM, 1), lambda i: (i, 0)),
        out_shape=jax.ShapeDtypeStruct((B, 1), jnp.float32),
    )(xu, xm, W1, b1.reshape(1, H1), W2, b2.reshape(1, H2),
      Wout, bout.reshape(1, 1))


def kernel(inputs, user_emb, movie_emb, W1, b1, W2, b2, Wout, bout):
    B = inputs.shape[0]
    DU = user_emb.shape[1]
    DM = movie_emb.shape[1]
    info = plsc.get_sparse_core_info()
    NC, NS = info.num_cores, info.num_subcores
    uidx = inputs[:, 0].reshape(B // _CHUNK, _CHUNK)
    midx = inputs[:, 1].reshape(B // _CHUNK, _CHUNK)
    xu, xm = _make_gather(B, DU, DM, NC, NS)(uidx, midx, user_emb, movie_emb)
    return _run_mlp(xu, xm, W1, b1, W2, b2, Wout, bout)
